# baseline (speedup 1.0000x reference)
import functools

import jax
import jax.numpy as jnp
from jax import lax
from jax.experimental import pallas as pl
from jax.experimental.pallas import tpu as pltpu
from jax.experimental.pallas import tpu_sc as plsc

_VOCAB = 100000
_D = 128
_CTX = 20
_B = 1024
_MAX_NORM = 1.0


def _sc_gather(idx_flat, table):
    n = idx_flat.shape[0]
    info = plsc.get_sparse_core_info()
    nw = info.num_cores * info.num_subcores
    per_w = n // nw
    mesh = plsc.VectorSubcoreMesh(core_axis_name="c", subcore_axis_name="s")

    @functools.partial(
        pl.kernel,
        mesh=mesh,
        out_type=jax.ShapeDtypeStruct((n, _D), jnp.float32),
        scratch_types=[
            pltpu.VMEM((per_w,), jnp.int32),
            pltpu.VMEM((per_w, _D), jnp.float32),
            pltpu.SemaphoreType.DMA,
        ],
    )
    def k(idx_hbm, table_hbm, out_hbm, idx_v, rows_v, sem):
        wid = lax.axis_index("s") * info.num_cores + lax.axis_index("c")
        base = wid * per_w
        pltpu.sync_copy(idx_hbm.at[pl.ds(base, per_w)], idx_v)
        pltpu.async_copy(table_hbm.at[idx_v], rows_v, sem).wait()
        pltpu.sync_copy(rows_v, out_hbm.at[pl.ds(base, per_w)])

    return k(idx_flat, table)


def _pool_body(g_ref, o_ref):
    x = g_ref[...]
    ss = jnp.sum(x * x, axis=-1, keepdims=True)
    norm = jnp.sqrt(ss)
    scale = jnp.where(norm > _MAX_NORM, _MAX_NORM / (norm + 1e-7), 1.0)
    o_ref[...] = jnp.sum(x * scale, axis=1) * (1.0 / _CTX)


def _pool(gathered):
    bt = 256
    return pl.pallas_call(
        _pool_body,
        grid=(_B // bt,),
        in_specs=[pl.BlockSpec((bt, _CTX, _D), lambda i: (i, 0, 0))],
        out_specs=pl.BlockSpec((bt, _D), lambda i: (i, 0)),
        out_shape=jax.ShapeDtypeStruct((_B, _D), jnp.float32),
        compiler_params=pltpu.CompilerParams(
            dimension_semantics=("arbitrary",)),
    )(gathered)


_TN = 2048
_NSTEPS = 49
_ND = 10
_DIRECT = _ND * _TN
_SCRW = _VOCAB - _DIRECT
_WCHUNK = 1792
_SCRPAD = 79872
_NBUF = 4


def _mm_body(p_ref, w_ref, b_ref, o_hbm, scr, ring, sems):
    j = pl.program_id(0)
    rb = lax.rem(j, _NBUF)
    acc = lax.dot_general(
        p_ref[...], w_ref[...],
        (((1,), (1,)), ((), ())),
        preferred_element_type=jnp.float32,
    )

    @pl.when(j >= _NBUF)
    def _():
        pltpu.make_async_copy(
            ring.at[rb], scr.at[:, pl.ds(0, _TN)], sems.at[rb],
        ).wait()

    ring[rb] = acc + b_ref[...][None, :]

    @pl.when(j < _ND)
    def _():
        pltpu.make_async_copy(
            ring.at[rb],
            o_hbm.at[:, pl.ds(j * _TN, _TN)],
            sems.at[rb],
        ).start()

    @pl.when(jnp.logical_and(j >= _ND, j < _NSTEPS - 1))
    def _():
        pltpu.make_async_copy(
            ring.at[rb],
            scr.at[:, pl.ds((j - _ND) * _TN, _TN)],
            sems.at[rb],
        ).start()

    @pl.when(j == _NSTEPS - 1)
    def _():
        last = pltpu.make_async_copy(
            ring.at[rb, :, pl.ds(0, _WCHUNK)],
            scr.at[:, pl.ds((_NSTEPS - 1 - _ND) * _TN, _WCHUNK)],
            sems.at[rb],
        )
        last.start()
        last.wait()
        for k in range(1, _NBUF):
            rbk = lax.rem(j - k, _NBUF)
            pltpu.make_async_copy(
                ring.at[rbk], scr.at[:, pl.ds(0, _TN)], sems.at[rbk],
            ).wait()


def _matmul(pooled, W, b):
    b_pad = jnp.pad(b, (0, _NSTEPS * _TN - _VOCAB))
    main, scr = pl.pallas_call(
        _mm_body,
        grid=(_NSTEPS,),
        in_specs=[
            pl.BlockSpec((_B, _D), lambda i: (0, 0)),
            pl.BlockSpec((_TN, _D), lambda i: (i, 0)),
            pl.BlockSpec((_TN,), lambda i: (i,)),
        ],
        out_specs=[
            pl.BlockSpec(memory_space=pl.ANY),
            pl.BlockSpec(memory_space=pl.ANY),
        ],
        out_shape=[
            jax.ShapeDtypeStruct((_B, _VOCAB), jnp.float32),
            jax.ShapeDtypeStruct((_B, _SCRPAD), jnp.float32),
        ],
        scratch_shapes=[
            pltpu.VMEM((_NBUF, _B, _TN), jnp.float32),
            pltpu.SemaphoreType.DMA((_NBUF,)),
        ],
        compiler_params=pltpu.CompilerParams(
            dimension_semantics=("arbitrary",)),
    )(pooled, W, b_pad)
    return jnp.concatenate([main[:, :_DIRECT], scr[:, :_SCRW]], axis=1)


def kernel(inputs, table, W, b):
    idx_flat = inputs.reshape(-1).astype(jnp.int32)
    gathered = _sc_gather(idx_flat, table)
    pooled = _pool(gathered.reshape(_B, _CTX, _D))
    return _matmul(pooled, W, b)


# speedup vs baseline: 1.3388x; 1.3388x over previous
import functools

import jax
import jax.numpy as jnp
from jax import lax
from jax.experimental import pallas as pl
from jax.experimental.pallas import tpu as pltpu
from jax.experimental.pallas import tpu_sc as plsc

_VOCAB = 100000
_D = 128
_CTX = 20
_B = 1024
_MAX_NORM = 1.0


def _sc_gather(idx_flat, table):
    n = idx_flat.shape[0]
    info = plsc.get_sparse_core_info()
    nw = info.num_cores * info.num_subcores
    per_w = n // nw
    mesh = plsc.VectorSubcoreMesh(core_axis_name="c", subcore_axis_name="s")

    @functools.partial(
        pl.kernel,
        mesh=mesh,
        out_type=jax.ShapeDtypeStruct((n, _D), jnp.float32),
        scratch_types=[
            pltpu.VMEM((per_w,), jnp.int32),
            pltpu.VMEM((per_w, _D), jnp.float32),
            pltpu.SemaphoreType.DMA,
        ],
    )
    def k(idx_hbm, table_hbm, out_hbm, idx_v, rows_v, sem):
        wid = lax.axis_index("s") * info.num_cores + lax.axis_index("c")
        base = wid * per_w
        pltpu.sync_copy(idx_hbm.at[pl.ds(base, per_w)], idx_v)
        pltpu.async_copy(table_hbm.at[idx_v], rows_v, sem).wait()
        pltpu.sync_copy(rows_v, out_hbm.at[pl.ds(base, per_w)])

    return k(idx_flat, table)


def _pool_body(g_ref, o_ref):
    x = g_ref[...]
    ss = jnp.sum(x * x, axis=-1, keepdims=True)
    norm = jnp.sqrt(ss)
    scale = jnp.where(norm > _MAX_NORM, _MAX_NORM / (norm + 1e-7), 1.0)
    o_ref[...] = jnp.sum(x * scale, axis=1) * (1.0 / _CTX)


def _pool(gathered):
    bt = 256
    return pl.pallas_call(
        _pool_body,
        grid=(_B // bt,),
        in_specs=[pl.BlockSpec((bt, _CTX, _D), lambda i: (i, 0, 0))],
        out_specs=pl.BlockSpec((bt, _D), lambda i: (i, 0)),
        out_shape=jax.ShapeDtypeStruct((_B, _D), jnp.float32),
        compiler_params=pltpu.CompilerParams(
            dimension_semantics=("arbitrary",)),
    )(gathered)


_TN = 2048
_NSTEPS = 49
_SCRPAD = _NSTEPS * _TN
_NBUF = 4


def _mm_body(p_ref, w_ref, b_ref, scr, ring, sems):
    j = pl.program_id(0)
    rb = lax.rem(j, _NBUF)
    acc = lax.dot_general(
        p_ref[...], w_ref[...],
        (((1,), (1,)), ((), ())),
        preferred_element_type=jnp.float32,
    )

    @pl.when(j >= _NBUF)
    def _():
        pltpu.make_async_copy(
            ring.at[rb], scr.at[:, pl.ds(0, _TN)], sems.at[rb],
        ).wait()

    ring[rb] = acc + b_ref[...][None, :]

    pltpu.make_async_copy(
        ring.at[rb],
        scr.at[:, pl.ds(j * _TN, _TN)],
        sems.at[rb],
    ).start()

    @pl.when(j == _NSTEPS - 1)
    def _():
        for k in range(_NBUF):
            rbk = lax.rem(j - k, _NBUF)
            pltpu.make_async_copy(
                ring.at[rbk], scr.at[:, pl.ds(0, _TN)], sems.at[rbk],
            ).wait()


def _matmul(pooled, W, b):
    b_pad = jnp.pad(b, (0, _NSTEPS * _TN - _VOCAB))
    scr = pl.pallas_call(
        _mm_body,
        grid=(_NSTEPS,),
        in_specs=[
            pl.BlockSpec((_B, _D), lambda i: (0, 0)),
            pl.BlockSpec((_TN, _D), lambda i: (i, 0)),
            pl.BlockSpec((_TN,), lambda i: (i,)),
        ],
        out_specs=pl.BlockSpec(memory_space=pl.ANY),
        out_shape=jax.ShapeDtypeStruct((_B, _SCRPAD), jnp.float32),
        scratch_shapes=[
            pltpu.VMEM((_NBUF, _B, _TN), jnp.float32),
            pltpu.SemaphoreType.DMA((_NBUF,)),
        ],
        compiler_params=pltpu.CompilerParams(
            dimension_semantics=("arbitrary",)),
    )(pooled, W, b_pad)
    return scr[:, :_VOCAB]


def kernel(inputs, table, W, b):
    idx_flat = inputs.reshape(-1).astype(jnp.int32)
    gathered = _sc_gather(idx_flat, table)
    pooled = _pool(gathered.reshape(_B, _CTX, _D))
    return _matmul(pooled, W, b)


# speedup vs baseline: 2.0581x; 1.5373x over previous
import functools

import jax
import jax.numpy as jnp
from jax import lax
from jax.experimental import pallas as pl
from jax.experimental.pallas import tpu as pltpu
from jax.experimental.pallas import tpu_sc as plsc

_VOCAB = 100000
_D = 128
_CTX = 20
_B = 1024
_MAX_NORM = 1.0


def _sc_gather(idx_flat, table):
    n = idx_flat.shape[0]
    info = plsc.get_sparse_core_info()
    nw = info.num_cores * info.num_subcores
    per_w = n // nw
    mesh = plsc.VectorSubcoreMesh(core_axis_name="c", subcore_axis_name="s")

    @functools.partial(
        pl.kernel,
        mesh=mesh,
        out_type=jax.ShapeDtypeStruct((n, _D), jnp.float32),
        scratch_types=[
            pltpu.VMEM((per_w,), jnp.int32),
            pltpu.VMEM((per_w, _D), jnp.float32),
            pltpu.SemaphoreType.DMA,
        ],
    )
    def k(idx_hbm, table_hbm, out_hbm, idx_v, rows_v, sem):
        wid = lax.axis_index("s") * info.num_cores + lax.axis_index("c")
        base = wid * per_w
        pltpu.sync_copy(idx_hbm.at[pl.ds(base, per_w)], idx_v)
        pltpu.async_copy(table_hbm.at[idx_v], rows_v, sem).wait()
        pltpu.sync_copy(rows_v, out_hbm.at[pl.ds(base, per_w)])

    return k(idx_flat, table)


def _pool_body(g_ref, o_ref):
    x = g_ref[...]
    ss = jnp.sum(x * x, axis=-1, keepdims=True)
    norm = jnp.sqrt(ss)
    scale = jnp.where(norm > _MAX_NORM, _MAX_NORM / (norm + 1e-7), 1.0)
    o_ref[...] = jnp.sum(x * scale, axis=1) * (1.0 / _CTX)


def _pool(gathered):
    bt = 256
    return pl.pallas_call(
        _pool_body,
        grid=(_B // bt,),
        in_specs=[pl.BlockSpec((bt, _CTX, _D), lambda i: (i, 0, 0))],
        out_specs=pl.BlockSpec((bt, _D), lambda i: (i, 0)),
        out_shape=jax.ShapeDtypeStruct((_B, _D), jnp.float32),
        compiler_params=pltpu.CompilerParams(
            dimension_semantics=("arbitrary",)),
    )(gathered)


_TN = 2048
_NSTEPS = 49
_ALIGNED = 99840
_WLAST = _ALIGNED - (_NSTEPS - 1) * _TN
_TAIL = _VOCAB - _ALIGNED
_NBUF = 4


def _mm_body(p_ref, w_ref, b_ref, o_hbm, ring, sems):
    j = pl.program_id(0)
    rb = lax.rem(j, _NBUF)
    acc = lax.dot_general(
        p_ref[...], w_ref[...],
        (((1,), (1,)), ((), ())),
        preferred_element_type=jnp.float32,
    )

    @pl.when(j >= _NBUF)
    def _():
        pltpu.make_async_copy(
            ring.at[rb],
            o_hbm.at[:, pl.ds((j - _NBUF) * _TN, _TN)],
            sems.at[rb],
        ).wait()

    ring[rb] = acc + b_ref[...][None, :]

    @pl.when(j < _NSTEPS - 1)
    def _():
        pltpu.make_async_copy(
            ring.at[rb],
            o_hbm.at[:, pl.ds(j * _TN, _TN)],
            sems.at[rb],
        ).start()

    @pl.when(j == _NSTEPS - 1)
    def _():
        last = pltpu.make_async_copy(
            ring.at[rb, :, pl.ds(0, _WLAST)],
            o_hbm.at[:, pl.ds(j * _TN, _WLAST)],
            sems.at[rb],
        )
        last.start()
        last.wait()
        for k in range(1, _NBUF):
            jj = j - k
            rbk = lax.rem(jj, _NBUF)
            pltpu.make_async_copy(
                ring.at[rbk],
                o_hbm.at[:, pl.ds(jj * _TN, _TN)],
                sems.at[rbk],
            ).wait()


def _tail_body(p_ref, w_ref, b_ref, o_ref):
    acc = lax.dot_general(
        p_ref[...], w_ref[...],
        (((1,), (1,)), ((), ())),
        preferred_element_type=jnp.float32,
    )
    o_ref[...] = acc + b_ref[...][None, :]


def _matmul(pooled, W, b):
    b_pad = jnp.pad(b, (0, _NSTEPS * _TN - _VOCAB))
    main = pl.pallas_call(
        _mm_body,
        grid=(_NSTEPS,),
        in_specs=[
            pl.BlockSpec((_B, _D), lambda i: (0, 0)),
            pl.BlockSpec((_TN, _D), lambda i: (i, 0)),
            pl.BlockSpec((_TN,), lambda i: (i,)),
        ],
        out_specs=pl.BlockSpec(memory_space=pl.ANY),
        out_shape=jax.ShapeDtypeStruct((_B, _VOCAB), jnp.float32),
        scratch_shapes=[
            pltpu.VMEM((_NBUF, _B, _TN), jnp.float32),
            pltpu.SemaphoreType.DMA((_NBUF,)),
        ],
        compiler_params=pltpu.CompilerParams(
            dimension_semantics=("arbitrary",)),
    )(pooled, W, b_pad)
    tail = pl.pallas_call(
        _tail_body,
        out_shape=jax.ShapeDtypeStruct((_B, _TAIL), jnp.float32),
    )(pooled, W[_ALIGNED:], b[_ALIGNED:])
    return lax.dynamic_update_slice(main, tail, (0, _ALIGNED))


def kernel(inputs, table, W, b):
    idx_flat = inputs.reshape(-1).astype(jnp.int32)
    gathered = _sc_gather(idx_flat, table)
    pooled = _pool(gathered.reshape(_B, _CTX, _D))
    return _matmul(pooled, W, b)
